# baseline (speedup 1.0000x reference)
import functools

import jax
import jax.numpy as jnp
from jax import lax
from jax.experimental import pallas as pl
from jax.experimental.pallas import tpu as pltpu
from jax.experimental.pallas import tpu_sc as plsc

NC = 2
NS = 16
NW = NC * NS
LANES = 16
CH = 832
CH3 = CH * 3
GG = 104
PW = CH * 16


def _mesh():
    return plsc.VectorSubcoreMesh(
        core_axis_name="c", subcore_axis_name="s", num_cores=NC,
        num_subcores=NS)


def _make_table_builder(N, CW, CWL):
    @functools.partial(
        pl.kernel,
        out_type=jax.ShapeDtypeStruct((2 * N, LANES), jnp.float32),
        mesh=_mesh(),
        scratch_types=[
            pltpu.VMEM((CW,), jnp.float32),
            pltpu.VMEM((CW, LANES), jnp.float32),
            pltpu.VMEM((CW, LANES), jnp.float32),
        ],
        compiler_params=pltpu.CompilerParams(use_tc_tiling_on_sc=False, needs_layout_passes=False),
    )
    def build(v_hbm, tbl_hbm, vrow, ta, tb):
        wid = lax.axis_index("c") * NS + lax.axis_index("s")
        iota = lax.iota(jnp.int32, LANES)

        def do(c0, cw):
            for b in range(LANES):
                pltpu.sync_copy(v_hbm.at[b, pl.ds(c0, cw)],
                                vrow.at[pl.ds(0, cw)])
                colb = iota * 0 + b

                def tbody(g, carry):
                    o = g * LANES
                    x = vrow[pl.ds(o, LANES)]
                    rows = o + iota
                    plsc.store_scatter(ta, [rows, colb], 0.5 - 0.5 * x)
                    plsc.store_scatter(tb, [rows, colb], 0.5 + 0.5 * x)
                    return carry

                lax.fori_loop(0, cw // LANES, tbody, 0)
            pltpu.sync_copy(ta.at[pl.ds(0, cw)], tbl_hbm.at[pl.ds(c0, cw)])
            pltpu.sync_copy(tb.at[pl.ds(0, cw)], tbl_hbm.at[pl.ds(N + c0, cw)])

        @pl.when(wid < NW - 1)
        def _():
            do(wid * CW, CW)

        @pl.when(wid == NW - 1)
        def _():
            do((NW - 1) * CW, CWL)

    return build


def _make_main(N, M):
    n_tail = M - NW * PW
    assert 0 <= n_tail and n_tail % 8 == 0 and n_tail <= CH

    @functools.partial(
        pl.kernel,
        out_type=jax.ShapeDtypeStruct((LANES, M), jnp.float32),
        mesh=_mesh(),
        scratch_types=[
            pltpu.VMEM((2, CH3), jnp.int32),
            pltpu.VMEM((2, CH3), jnp.float32),
            pltpu.VMEM((2, CH3, LANES), jnp.float32),
            pltpu.VMEM((2, LANES, CH), jnp.float32),
            pltpu.SemaphoreType.DMA,
            pltpu.SemaphoreType.DMA,
            pltpu.SemaphoreType.DMA,
            pltpu.SemaphoreType.DMA,
        ],
        compiler_params=pltpu.CompilerParams(use_tc_tiling_on_sc=False, needs_layout_passes=False),
    )
    def main(tbl, idxf, sgnf, out, idxv, sgnv, gbuf, obuf,
             gsem0, gsem1, osem0, osem1):
        gsem = (gsem0, gsem1)
        osem = (osem0, osem1)
        wid = lax.axis_index("c") * NS + lax.axis_index("s")
        wbase = wid * PW
        iota = lax.iota(jnp.int32, LANES)

        def load_fire(ci, p):
            base3 = (wbase + ci * CH) * 3
            pltpu.sync_copy(idxf.at[pl.ds(base3, CH3)], idxv.at[p])
            pltpu.sync_copy(sgnf.at[pl.ds(base3, CH3)], sgnv.at[p])

            def abody(g, carry):
                o = g * 64
                for u in range(4):
                    oo = o + u * LANES
                    ii = idxv[p, pl.ds(oo, LANES)]
                    ss = sgnv[p, pl.ds(oo, LANES)]
                    idxv[p, pl.ds(oo, LANES)] = ii + jnp.where(
                        ss < 0.0, jnp.int32(N), jnp.int32(0))
                return carry

            lax.fori_loop(0, CH3 // 64, abody, 0)
            for j in range(CH3 // GG):
                pltpu.async_copy(
                    tbl.at[idxv.at[p, pl.ds(j * GG, GG)]],
                    gbuf.at[p, pl.ds(j * GG, GG)],
                    gsem[p])

        def wait_gather(p):
            pltpu.make_async_copy(
                tbl.at[pl.ds(0, CH3)], gbuf.at[p], gsem[p]).wait()

        def compute(p):
            def cbody(i, carry):
                c = i * 4
                for u in range(4):
                    r = 3 * (c + u)
                    m = jnp.minimum(
                        jnp.minimum(gbuf[p, r], gbuf[p, r + 1]),
                        gbuf[p, r + 2])
                    plsc.store_scatter(
                        obuf.at[p], [iota, iota * 0 + (c + u)], m)
                return carry

            lax.fori_loop(0, CH // 4, cbody, 0)

        def flush_out(ci, p):
            base = wbase + ci * CH
            for b in range(LANES):
                pltpu.async_copy(
                    obuf.at[p, b], out.at[b, pl.ds(base, CH)], osem[p])

        def wait_out(p):
            pltpu.make_async_copy(
                obuf.at[p],
                out.at[pl.ds(0, LANES), pl.ds(0, CH)],
                osem[p]).wait()

        def step(ci, p, do_wait_out, next_ci):
            wait_gather(p)
            if do_wait_out:
                wait_out(p)
            compute(p)
            flush_out(ci, p)
            if next_ci is not None:
                load_fire(next_ci, p)

        load_fire(0, 0)
        load_fire(1, 1)
        step(0, 0, False, 2)
        step(1, 1, False, 3)

        def pair(t, carry):
            ca = 2 * t
            step(ca, 0, True, ca + 2)
            step(ca + 1, 1, True, ca + 3)
            return carry

        lax.fori_loop(1, 7, pair, 0)
        step(14, 0, True, None)
        step(15, 1, True, None)
        wait_out(0)
        wait_out(1)

        if n_tail:
            @pl.when(wid == 0)
            def _():
                base = NW * PW
                base3 = base * 3
                t3 = n_tail * 3
                pltpu.sync_copy(idxf.at[pl.ds(base3, t3)],
                                idxv.at[0, pl.ds(0, t3)])
                pltpu.sync_copy(sgnf.at[pl.ds(base3, t3)],
                                sgnv.at[0, pl.ds(0, t3)])

                def abody(g, carry):
                    o = g * LANES
                    ii = idxv[0, pl.ds(o, LANES)]
                    ss = sgnv[0, pl.ds(o, LANES)]
                    idxv[0, pl.ds(o, LANES)] = ii + jnp.where(
                        ss < 0.0, jnp.int32(N), jnp.int32(0))
                    return carry

                lax.fori_loop(0, t3 // LANES, abody, 0)
                pltpu.async_copy(
                    tbl.at[idxv.at[0, pl.ds(0, t3)]],
                    gbuf.at[0, pl.ds(0, t3)], gsem0)
                pltpu.make_async_copy(
                    tbl.at[pl.ds(0, t3)],
                    gbuf.at[0, pl.ds(0, t3)], gsem0).wait()

                def cbody(i, carry):
                    r = 3 * i
                    m = jnp.minimum(
                        jnp.minimum(gbuf[0, r], gbuf[0, r + 1]),
                        gbuf[0, r + 2])
                    plsc.store_scatter(obuf.at[0], [iota, iota * 0 + i], m)
                    return carry

                lax.fori_loop(0, n_tail, cbody, 0)
                for b in range(LANES):
                    pltpu.async_copy(
                        obuf.at[0, b, pl.ds(0, n_tail)],
                        out.at[b, pl.ds(base, n_tail)], osem0)
                pltpu.make_async_copy(
                    obuf.at[0, pl.ds(0, LANES), pl.ds(0, n_tail)],
                    out.at[pl.ds(0, LANES), pl.ds(0, n_tail)],
                    osem0).wait()

    return main


def kernel(v, input_idx, input_sign):
    B, N = v.shape
    M, K = input_idx.shape
    assert B == LANES and K == 3

    CW = ((N + NW - 1) // NW + LANES - 1) // LANES * LANES
    CWL = N - (NW - 1) * CW
    assert CWL > 0 and CWL % LANES == 0 and N % 8 == 0

    tbl = _make_table_builder(N, CW, CWL)(v)
    out = _make_main(N, M)(tbl, input_idx.reshape(-1),
                           input_sign.reshape(-1))
    return out


# speedup vs baseline: 3.0379x; 3.0379x over previous
import functools

import jax
import jax.numpy as jnp
from jax import lax
from jax.experimental import pallas as pl
from jax.experimental.pallas import tpu as pltpu
from jax.experimental.pallas import tpu_sc as plsc

NC = 2
NS = 16
NW = NC * NS
LANES = 16
CH = 832
CH3 = CH * 3
GG = 104
NCHUNK = 16
PW = CH * NCHUNK


def _mesh():
    return plsc.VectorSubcoreMesh(
        core_axis_name="c", subcore_axis_name="s", num_cores=NC,
        num_subcores=NS)


def _params():
    return pltpu.CompilerParams(
        use_tc_tiling_on_sc=False, needs_layout_passes=False)


def _make_table_builder(NP, RW):
    SB = RW // 2

    @functools.partial(
        pl.kernel,
        out_type=jax.ShapeDtypeStruct((2 * NP, LANES), jnp.float32),
        mesh=_mesh(),
        scratch_types=[
            pltpu.VMEM((SB, LANES), jnp.float32),
            pltpu.VMEM((SB, LANES), jnp.float32),
        ],
        compiler_params=_params(),
    )
    def build(vt_hbm, tbl_hbm, vbuf, tbuf):
        wid = lax.axis_index("c") * NS + lax.axis_index("s")
        r0 = wid * RW
        for h in range(2):
            base = r0 + h * SB
            pltpu.sync_copy(vt_hbm.at[pl.ds(base, SB)], vbuf)

            def pa(i, carry):
                r = i * 4
                for u in range(4):
                    tbuf[r + u] = 0.5 - 0.5 * vbuf[r + u]
                return carry

            lax.fori_loop(0, SB // 4, pa, 0)
            pltpu.sync_copy(tbuf, tbl_hbm.at[pl.ds(base, SB)])

            def pb(i, carry):
                r = i * 4
                for u in range(4):
                    tbuf[r + u] = 0.5 + 0.5 * vbuf[r + u]
                return carry

            lax.fori_loop(0, SB // 4, pb, 0)
            pltpu.sync_copy(tbuf, tbl_hbm.at[pl.ds(NP + base, SB)])

    return build


def _make_main(NP, M):
    n_tail = M - NW * PW
    assert 0 <= n_tail <= CH and n_tail % LANES == 0

    @functools.partial(
        pl.kernel,
        out_type=jax.ShapeDtypeStruct((M, LANES), jnp.float32),
        mesh=_mesh(),
        scratch_types=[
            pltpu.VMEM((2, 3, CH), jnp.int32),
            pltpu.VMEM((2, 3, CH), jnp.float32),
            pltpu.VMEM((2, 3, CH, LANES), jnp.float32),
            pltpu.VMEM((2, CH, LANES), jnp.float32),
            pltpu.SemaphoreType.DMA,
            pltpu.SemaphoreType.DMA,
            pltpu.SemaphoreType.DMA,
            pltpu.SemaphoreType.DMA,
        ],
        compiler_params=_params(),
    )
    def main(tbl, i0, i1, i2, s0, s1, s2, out, idxv, sgnv, gbuf, obuf,
             gsem0, gsem1, osem0, osem1):
        gsem = (gsem0, gsem1)
        osem = (osem0, osem1)
        irefs = (i0, i1, i2)
        srefs = (s0, s1, s2)
        wid = lax.axis_index("c") * NS + lax.axis_index("s")
        wbase = wid * PW

        def load_fire(ci, p):
            base = wbase + ci * CH
            for k in range(3):
                pltpu.sync_copy(irefs[k].at[pl.ds(base, CH)], idxv.at[p, k])
                pltpu.sync_copy(srefs[k].at[pl.ds(base, CH)], sgnv.at[p, k])

            def abody(g, carry):
                o = g * 64
                for k in range(3):
                    for u in range(4):
                        oo = o + u * LANES
                        ii = idxv[p, k, pl.ds(oo, LANES)]
                        ss = sgnv[p, k, pl.ds(oo, LANES)]
                        idxv[p, k, pl.ds(oo, LANES)] = ii + jnp.where(
                            ss < 0.0, jnp.int32(NP), jnp.int32(0))
                return carry

            lax.fori_loop(0, CH // 64, abody, 0)
            for k in range(3):
                for j in range(CH // GG):
                    pltpu.async_copy(
                        tbl.at[idxv.at[p, k, pl.ds(j * GG, GG)]],
                        gbuf.at[p, k, pl.ds(j * GG, GG)],
                        gsem[p])

        def wait_gather(p):
            for k in range(3):
                pltpu.make_async_copy(
                    tbl.at[pl.ds(0, CH)], gbuf.at[p, k], gsem[p]).wait()

        def compute(p):
            def cbody(i, carry):
                c = i * 4
                for u in range(4):
                    obuf[p, c + u] = jnp.minimum(
                        jnp.minimum(gbuf[p, 0, c + u], gbuf[p, 1, c + u]),
                        gbuf[p, 2, c + u])
                return carry

            lax.fori_loop(0, CH // 4, cbody, 0)

        def flush_out(ci, p):
            pltpu.async_copy(
                obuf.at[p], out.at[pl.ds(wbase + ci * CH, CH)], osem[p])

        def wait_out(p):
            pltpu.make_async_copy(
                obuf.at[p], out.at[pl.ds(0, CH)], osem[p]).wait()

        def step(ci, p, do_wait_out, next_ci):
            wait_gather(p)
            if do_wait_out:
                wait_out(p)
            compute(p)
            flush_out(ci, p)
            if next_ci is not None:
                load_fire(next_ci, p)

        load_fire(0, 0)
        load_fire(1, 1)
        step(0, 0, False, 2)
        step(1, 1, False, 3)

        def pair(t, carry):
            ca = 2 * t
            step(ca, 0, True, ca + 2)
            step(ca + 1, 1, True, ca + 3)
            return carry

        lax.fori_loop(1, NCHUNK // 2 - 1, pair, 0)
        step(NCHUNK - 2, 0, True, None)
        step(NCHUNK - 1, 1, True, None)
        wait_out(0)
        wait_out(1)

        if n_tail:
            @pl.when(wid == 0)
            def _():
                base = NW * PW
                for k in range(3):
                    pltpu.sync_copy(irefs[k].at[pl.ds(base, n_tail)],
                                    idxv.at[0, k, pl.ds(0, n_tail)])
                    pltpu.sync_copy(srefs[k].at[pl.ds(base, n_tail)],
                                    sgnv.at[0, k, pl.ds(0, n_tail)])

                def abody(g, carry):
                    o = g * LANES
                    for k in range(3):
                        ii = idxv[0, k, pl.ds(o, LANES)]
                        ss = sgnv[0, k, pl.ds(o, LANES)]
                        idxv[0, k, pl.ds(o, LANES)] = ii + jnp.where(
                            ss < 0.0, jnp.int32(NP), jnp.int32(0))
                    return carry

                lax.fori_loop(0, n_tail // LANES, abody, 0)
                for k in range(3):
                    pltpu.async_copy(
                        tbl.at[idxv.at[0, k, pl.ds(0, n_tail)]],
                        gbuf.at[0, k, pl.ds(0, n_tail)], gsem0)
                for k in range(3):
                    pltpu.make_async_copy(
                        tbl.at[pl.ds(0, n_tail)],
                        gbuf.at[0, k, pl.ds(0, n_tail)], gsem0).wait()

                def cbody(i, carry):
                    obuf[0, i] = jnp.minimum(
                        jnp.minimum(gbuf[0, 0, i], gbuf[0, 1, i]),
                        gbuf[0, 2, i])
                    return carry

                lax.fori_loop(0, n_tail, cbody, 0)
                pltpu.async_copy(
                    obuf.at[0, pl.ds(0, n_tail)],
                    out.at[pl.ds(base, n_tail)], osem0)
                pltpu.make_async_copy(
                    obuf.at[0, pl.ds(0, n_tail)],
                    out.at[pl.ds(base, n_tail)], osem0).wait()

    return main


def kernel(v, input_idx, input_sign):
    B, N = v.shape
    M, K = input_idx.shape
    assert B == LANES and K == 3

    NP = (N + NW * 8 - 1) // (NW * 8) * (NW * 8)
    RW = NP // NW
    assert RW % 8 == 0

    vt = jnp.zeros((NP, LANES), jnp.float32).at[:N].set(v.T)
    tbl = _make_table_builder(NP, RW)(vt)
    outT = _make_main(NP, M)(
        tbl,
        input_idx[:, 0], input_idx[:, 1], input_idx[:, 2],
        input_sign[:, 0], input_sign[:, 1], input_sign[:, 2])
    return outT.T
